# baseline (speedup 1.0000x reference)
import jax
import jax.numpy as jnp
from jax import lax
from jax.experimental import pallas as pl
from jax.experimental.pallas import tpu as pltpu
from jax.experimental.pallas import tpu_sc as plsc

N = 10000
E = 320000
D = 128

NC = 2
NS = 16
NW = NC * NS
K = 128
EP = E // NW
G = 80
EPP = G * K
HALF = 5120
AR = 5376
RPT = AR // NS
ZR = 24


def _matmul_body(h_ref, w_ref, b_ref, o_ref):
    o_ref[...] = (
        jnp.dot(h_ref[...], w_ref[...], preferred_element_type=jnp.float32)
        + b_ref[...]
    )


def _combine_body(p_ref, o_ref):
    o_ref[...] = jnp.maximum(p_ref[0, 0] + p_ref[1, 0], 0.0)


def _sc_body(hw, src, dst, av, out, src_v, dst_v, dstp_v, av_v, rows_a,
             rows_b, zbuf, acc, gsa, gsb, ssa, ssb):
    c = lax.axis_index("c")
    s = lax.axis_index("s")
    wid = c * NS + s

    def _zero_row(i, _):
        for j in range(D // 16):
            zbuf[i, pl.ds(j * 16, 16)] = jnp.zeros((16,), jnp.float32)
        return 0

    lax.fori_loop(0, ZR, _zero_row, 0)

    pltpu.sync_copy(src.at[wid], src_v)
    pltpu.sync_copy(dst.at[wid], dst_v)
    pltpu.sync_copy(av.at[wid], av_v)

    dump = HALF + s * 16 + lax.iota(jnp.int32, 16)

    def _scale_buf(buf, g):
        @plsc.parallel_loop(0, K, step=1, unroll=4)
        def _scale(e):
            ab = plsc.load_gather(av_v, [jnp.full((16,), g * K + e, jnp.int32)])
            for j in range(D // 16):
                sl = pl.ds(j * 16, 16)
                buf[e, sl] = buf[e, sl] * ab

    for p in range(2):
        def _remap(r, _):
            for c5 in range(K // 16):
                sl = pl.ds(c5 * 16, 16)
                d16 = dst_v[r, sl]
                local = d16 - p * HALF
                oob = (local < 0) | (local >= HALF)
                dstp_v[r, sl] = jnp.where(oob, dump, local)
            return 0

        lax.fori_loop(0, G, _remap, 0)

        for r in range(RPT // ZR):
            pltpu.sync_copy(zbuf, acc.at[pl.ds(s * RPT + r * ZR, ZR)])
        plsc.subcore_barrier()

        pltpu.async_copy(hw.at[src_v.at[0]], rows_a, gsa)

        def _pair(t, _):
            g0 = 2 * t
            g1 = 2 * t + 1

            @pl.when(t > 0)
            def _():
                pltpu.make_async_copy(rows_b, acc.at[dstp_v.at[g1]], ssb).wait()

            pltpu.async_copy(hw.at[src_v.at[g1]], rows_b, gsb)

            pltpu.make_async_copy(hw.at[src_v.at[g0]], rows_a, gsa).wait()
            _scale_buf(rows_a, g0)
            pltpu.async_copy(rows_a, acc.at[dstp_v.at[g0]], ssa, add=True)

            pltpu.make_async_copy(hw.at[src_v.at[g1]], rows_b, gsb).wait()
            _scale_buf(rows_b, g1)
            pltpu.async_copy(rows_b, acc.at[dstp_v.at[g1]], ssb, add=True)

            pltpu.make_async_copy(rows_a, acc.at[dstp_v.at[g0]], ssa).wait()

            @pl.when(t < G // 2 - 1)
            def _():
                pltpu.async_copy(hw.at[src_v.at[g0 + 2]], rows_a, gsa)

            return 0

        lax.fori_loop(0, G // 2, _pair, 0)

        pltpu.make_async_copy(rows_b, acc.at[dstp_v.at[G - 1]], ssb).wait()
        plsc.subcore_barrier()

        pltpu.sync_copy(
            acc.at[pl.ds(s * RPT, RPT)],
            out.at[c, p, pl.ds(s * RPT, RPT)],
        )
        plsc.subcore_barrier()


def kernel(H, edge_index, A_vals, W, b):
    hw = pl.pallas_call(
        _matmul_body,
        grid=(10,),
        in_specs=[
            pl.BlockSpec((N // 10, D), lambda i: (i, 0)),
            pl.BlockSpec((D, D), lambda i: (0, 0)),
            pl.BlockSpec((1, D), lambda i: (0, 0)),
        ],
        out_specs=pl.BlockSpec((N // 10, D), lambda i: (i, 0)),
        out_shape=jax.ShapeDtypeStruct((N, D), jnp.float32),
    )(H, W, b.reshape(1, D))

    pad = ((0, 0), (0, EPP - EP))
    src2 = jnp.pad(edge_index[0].reshape(NW, EP), pad).reshape(NW, G, K)
    dst2 = jnp.pad(edge_index[1].reshape(NW, EP), pad).reshape(NW, G, K)
    av2 = jnp.pad(A_vals.reshape(NW, EP), pad)

    mesh = plsc.VectorSubcoreMesh(
        core_axis_name="c", subcore_axis_name="s", num_cores=NC, num_subcores=NS
    )
    scatter = pl.kernel(
        _sc_body,
        out_type=jax.ShapeDtypeStruct((NC, 2, AR, D), jnp.float32),
        mesh=mesh,
        compiler_params=pltpu.CompilerParams(needs_layout_passes=False),
        scratch_types=[
            pltpu.VMEM((G, K), jnp.int32),
            pltpu.VMEM((G, K), jnp.int32),
            pltpu.VMEM((G, K), jnp.int32),
            pltpu.VMEM((EPP,), jnp.float32),
            pltpu.VMEM((K, D), jnp.float32),
            pltpu.VMEM((K, D), jnp.float32),
            pltpu.VMEM((ZR, D), jnp.float32),
            pltpu.VMEM_SHARED((AR, D), jnp.float32),
            pltpu.SemaphoreType.DMA,
            pltpu.SemaphoreType.DMA,
            pltpu.SemaphoreType.DMA,
            pltpu.SemaphoreType.DMA,
        ],
    )
    partials = scatter(hw, src2, dst2, av2)

    out = pl.pallas_call(
        _combine_body,
        grid=(2, 10),
        in_specs=[
            pl.BlockSpec((NC, 1, 512, D), lambda q, r: (0, q, r, 0)),
        ],
        out_specs=pl.BlockSpec((512, D), lambda q, r: (q * 10 + r, 0)),
        out_shape=jax.ShapeDtypeStruct((N, D), jnp.float32),
    )(partials)
    return out


# speedup vs baseline: 2.4483x; 2.4483x over previous
import jax
import jax.numpy as jnp
from jax import lax
from jax.experimental import pallas as pl
from jax.experimental.pallas import tpu as pltpu
from jax.experimental.pallas import tpu_sc as plsc

N = 10000
E = 320000
D = 128

NC = 2
NS = 16
NW = NC * NS
K = 80
EP = E // NW
G = EP // K
HALF = 5120
AR = 5376
RPT = AR // NS
ZR = 24


def _matmul_body(h_ref, w_ref, b_ref, o_ref):
    o_ref[...] = (
        jnp.dot(h_ref[...], w_ref[...], preferred_element_type=jnp.float32)
        + b_ref[...]
    )


def _combine_body(p_ref, o_ref):
    o_ref[...] = jnp.maximum(p_ref[0, 0] + p_ref[1, 0], 0.0)


def _sc_body(hw, src, dst, av, out, src_v, dst_v, dstp_v, av_v, rows_a,
             rows_b, zbuf, acc, gsa, gsb, ssa, ssb):
    c = lax.axis_index("c")
    s = lax.axis_index("s")
    wid = c * NS + s

    def _zero_row(i, _):
        for j in range(D // 16):
            zbuf[i, pl.ds(j * 16, 16)] = jnp.zeros((16,), jnp.float32)
        return 0

    lax.fori_loop(0, ZR, _zero_row, 0)

    pltpu.sync_copy(src.at[wid], src_v)
    pltpu.sync_copy(dst.at[wid], dst_v)
    pltpu.sync_copy(av.at[wid], av_v)

    dump = HALF + s * 16 + lax.iota(jnp.int32, 16)

    def _scale_buf(buf, g):
        @plsc.parallel_loop(0, K, step=16, unroll=1)
        def _scale(e0):
            a16 = av_v[pl.ds(g * K + e0, 16)]
            for i in range(16):
                ab = lax.gather(
                    a16, jnp.full((16, 1), i, jnp.int32),
                    dimension_numbers=lax.GatherDimensionNumbers(
                        offset_dims=(), collapsed_slice_dims=(0,),
                        start_index_map=(0,)),
                    slice_sizes=(1,),
                    mode=lax.GatherScatterMode.PROMISE_IN_BOUNDS)
                for j in range(D // 16):
                    sl = pl.ds(j * 16, 16)
                    buf[e0 + i, sl] = buf[e0 + i, sl] * ab

    for p in range(2):
        def _remap(r, _):
            for c5 in range(K // 16):
                sl = pl.ds(c5 * 16, 16)
                d16 = dst_v[r, sl]
                local = d16 - p * HALF
                oob = (local < 0) | (local >= HALF)
                dstp_v[r, sl] = jnp.where(oob, dump, local)
            return 0

        lax.fori_loop(0, G, _remap, 0)

        for r in range(RPT // ZR):
            pltpu.sync_copy(zbuf, acc.at[pl.ds(s * RPT + r * ZR, ZR)])
        plsc.subcore_barrier()

        pltpu.async_copy(hw.at[src_v.at[0]], rows_a, gsa)

        def _pair(t, _):
            g0 = 2 * t
            g1 = 2 * t + 1

            @pl.when(t > 0)
            def _():
                pltpu.make_async_copy(rows_b, acc.at[dstp_v.at[g1]], ssb).wait()

            pltpu.async_copy(hw.at[src_v.at[g1]], rows_b, gsb)

            pltpu.make_async_copy(hw.at[src_v.at[g0]], rows_a, gsa).wait()
            _scale_buf(rows_a, g0)
            pltpu.async_copy(rows_a, acc.at[dstp_v.at[g0]], ssa, add=True)

            pltpu.make_async_copy(hw.at[src_v.at[g1]], rows_b, gsb).wait()
            _scale_buf(rows_b, g1)
            pltpu.async_copy(rows_b, acc.at[dstp_v.at[g1]], ssb, add=True)

            pltpu.make_async_copy(rows_a, acc.at[dstp_v.at[g0]], ssa).wait()
            pltpu.async_copy(hw.at[src_v.at[g0 + 2]], rows_a, gsa)
            return 0

        lax.fori_loop(0, G // 2, _pair, 0)

        pltpu.make_async_copy(rows_b, acc.at[dstp_v.at[G - 2]], ssb).wait()
        pltpu.make_async_copy(hw.at[src_v.at[G - 1]], rows_a, gsa).wait()
        _scale_buf(rows_a, G - 1)
        pltpu.sync_copy(rows_a, acc.at[dstp_v.at[G - 1]], add=True)
        plsc.subcore_barrier()

        pltpu.sync_copy(
            acc.at[pl.ds(s * (HALF // NS), HALF // NS)],
            out.at[c, p, pl.ds(s * (HALF // NS), HALF // NS)],
        )
        plsc.subcore_barrier()


def kernel(H, edge_index, A_vals, W, b):
    hw = pl.pallas_call(
        _matmul_body,
        grid=(10,),
        in_specs=[
            pl.BlockSpec((N // 10, D), lambda i: (i, 0)),
            pl.BlockSpec((D, D), lambda i: (0, 0)),
            pl.BlockSpec((1, D), lambda i: (0, 0)),
        ],
        out_specs=pl.BlockSpec((N // 10, D), lambda i: (i, 0)),
        out_shape=jax.ShapeDtypeStruct((N, D), jnp.float32),
    )(H, W, b.reshape(1, D))

    src2 = edge_index[0].reshape(NW, G, K)
    dst2 = edge_index[1].reshape(NW, G, K)
    av2 = A_vals.reshape(NW, G * K)

    mesh = plsc.VectorSubcoreMesh(
        core_axis_name="c", subcore_axis_name="s", num_cores=NC, num_subcores=NS
    )
    scatter = pl.kernel(
        _sc_body,
        out_type=jax.ShapeDtypeStruct((NC, 2, HALF, D), jnp.float32),
        mesh=mesh,
        compiler_params=pltpu.CompilerParams(needs_layout_passes=False),
        scratch_types=[
            pltpu.VMEM((G, K), jnp.int32),
            pltpu.VMEM((G, K), jnp.int32),
            pltpu.VMEM((G, K), jnp.int32),
            pltpu.VMEM((G * K,), jnp.float32),
            pltpu.VMEM((K, D), jnp.float32),
            pltpu.VMEM((K, D), jnp.float32),
            pltpu.VMEM((ZR, D), jnp.float32),
            pltpu.VMEM_SHARED((AR, D), jnp.float32),
            pltpu.SemaphoreType.DMA,
            pltpu.SemaphoreType.DMA,
            pltpu.SemaphoreType.DMA,
            pltpu.SemaphoreType.DMA,
        ],
    )
    partials = scatter(hw, src2, dst2, av2)

    out = pl.pallas_call(
        _combine_body,
        grid=(2, 10),
        in_specs=[
            pl.BlockSpec((NC, 1, 512, D), lambda q, r: (0, q, r, 0)),
        ],
        out_specs=pl.BlockSpec((512, D), lambda q, r: (q * 10 + r, 0)),
        out_shape=jax.ShapeDtypeStruct((N, D), jnp.float32),
    )(partials)
    return out


# speedup vs baseline: 2.5285x; 1.0328x over previous
import jax
import jax.numpy as jnp
from jax import lax
from jax.experimental import pallas as pl
from jax.experimental.pallas import tpu as pltpu
from jax.experimental.pallas import tpu_sc as plsc

N = 10000
E = 320000
D = 128

NC = 2
NS = 16
NW = NC * NS
K = 80
EP = E // NW
G = EP // K
HALF = 5120
AR = 5632
RPT = AR // NS
ZR = 32


def _matmul_body(h_ref, w_ref, b_ref, o_ref):
    o_ref[...] = (
        jnp.dot(h_ref[...], w_ref[...], preferred_element_type=jnp.float32)
        + b_ref[...]
    )


def _combine_body(p_ref, o_ref):
    o_ref[...] = jnp.maximum(p_ref[0, 0] + p_ref[1, 0], 0.0)


def _sc_body(hw, src, dst, av, out, src_v, dst_v, dstp_v, av_v, rows_a,
             rows_b, zbuf, acc, gsa, gsb, ssa, ssb):
    c = lax.axis_index("c")
    s = lax.axis_index("s")
    wid = c * NS + s

    def _zero_row(i, _):
        for j in range(D // 16):
            zbuf[i, pl.ds(j * 16, 16)] = jnp.zeros((16,), jnp.float32)
        return 0

    lax.fori_loop(0, ZR, _zero_row, 0)

    pltpu.sync_copy(src.at[wid], src_v)
    pltpu.sync_copy(dst.at[wid], dst_v)
    pltpu.sync_copy(av.at[wid], av_v)

    dbase = HALF + s * 32

    def _scale_buf(buf, g):
        @plsc.parallel_loop(0, K, step=1, unroll=4)
        def _scale(e):
            ab = plsc.load_gather(av_v, [jnp.full((16,), g * K + e, jnp.int32)])
            for j in range(D // 16):
                sl = pl.ds(j * 16, 16)
                buf[e, sl] = buf[e, sl] * ab

    for p in range(2):
        def _remap(r, _):
            for c5 in range(K // 16):
                sl = pl.ds(c5 * 16, 16)
                d16 = dst_v[r, sl]
                local = d16 - p * HALF
                oob = (local < 0) | (local >= HALF)
                dump = dbase + ((c5 * 16 + lax.iota(jnp.int32, 16)) & 31)
                dstp_v[r, sl] = jnp.where(oob, dump, local)
            return 0

        lax.fori_loop(0, G, _remap, 0)

        for r in range(RPT // ZR):
            pltpu.sync_copy(zbuf, acc.at[pl.ds(s * RPT + r * ZR, ZR)])
        plsc.subcore_barrier()

        pltpu.async_copy(hw.at[src_v.at[0]], rows_a, gsa)

        def _pair(t, _):
            g0 = 2 * t
            g1 = 2 * t + 1

            @pl.when(t > 0)
            def _():
                pltpu.make_async_copy(rows_b, acc.at[dstp_v.at[g1]], ssb).wait()

            pltpu.async_copy(hw.at[src_v.at[g1]], rows_b, gsb)

            pltpu.make_async_copy(hw.at[src_v.at[g0]], rows_a, gsa).wait()
            _scale_buf(rows_a, g0)
            pltpu.async_copy(rows_a, acc.at[dstp_v.at[g0]], ssa, add=True)

            pltpu.make_async_copy(hw.at[src_v.at[g1]], rows_b, gsb).wait()
            _scale_buf(rows_b, g1)
            pltpu.async_copy(rows_b, acc.at[dstp_v.at[g1]], ssb, add=True)

            pltpu.make_async_copy(rows_a, acc.at[dstp_v.at[g0]], ssa).wait()
            pltpu.async_copy(hw.at[src_v.at[g0 + 2]], rows_a, gsa)
            return 0

        lax.fori_loop(0, G // 2, _pair, 0)

        pltpu.make_async_copy(rows_b, acc.at[dstp_v.at[G - 2]], ssb).wait()
        pltpu.make_async_copy(hw.at[src_v.at[G - 1]], rows_a, gsa).wait()
        _scale_buf(rows_a, G - 1)
        pltpu.sync_copy(rows_a, acc.at[dstp_v.at[G - 1]], add=True)
        plsc.subcore_barrier()

        pltpu.sync_copy(
            acc.at[pl.ds(s * (HALF // NS), HALF // NS)],
            out.at[c, p, pl.ds(s * (HALF // NS), HALF // NS)],
        )
        plsc.subcore_barrier()


def kernel(H, edge_index, A_vals, W, b):
    hw = pl.pallas_call(
        _matmul_body,
        grid=(10,),
        in_specs=[
            pl.BlockSpec((N // 10, D), lambda i: (i, 0)),
            pl.BlockSpec((D, D), lambda i: (0, 0)),
            pl.BlockSpec((1, D), lambda i: (0, 0)),
        ],
        out_specs=pl.BlockSpec((N // 10, D), lambda i: (i, 0)),
        out_shape=jax.ShapeDtypeStruct((N, D), jnp.float32),
    )(H, W, b.reshape(1, D))

    src2 = edge_index[0].reshape(NW, G, K)
    dst2 = edge_index[1].reshape(NW, G, K)
    av2 = A_vals.reshape(NW, G * K)

    mesh = plsc.VectorSubcoreMesh(
        core_axis_name="c", subcore_axis_name="s", num_cores=NC, num_subcores=NS
    )
    scatter = pl.kernel(
        _sc_body,
        out_type=jax.ShapeDtypeStruct((NC, 2, HALF, D), jnp.float32),
        mesh=mesh,
        compiler_params=pltpu.CompilerParams(needs_layout_passes=False),
        scratch_types=[
            pltpu.VMEM((G, K), jnp.int32),
            pltpu.VMEM((G, K), jnp.int32),
            pltpu.VMEM((G, K), jnp.int32),
            pltpu.VMEM((G * K,), jnp.float32),
            pltpu.VMEM((K, D), jnp.float32),
            pltpu.VMEM((K, D), jnp.float32),
            pltpu.VMEM((ZR, D), jnp.float32),
            pltpu.VMEM_SHARED((AR, D), jnp.float32),
            pltpu.SemaphoreType.DMA,
            pltpu.SemaphoreType.DMA,
            pltpu.SemaphoreType.DMA,
            pltpu.SemaphoreType.DMA,
        ],
    )
    partials = scatter(hw, src2, dst2, av2)

    out = pl.pallas_call(
        _combine_body,
        grid=(2, 10),
        in_specs=[
            pl.BlockSpec((NC, 1, 512, D), lambda q, r: (0, q, r, 0)),
        ],
        out_specs=pl.BlockSpec((512, D), lambda q, r: (q * 10 + r, 0)),
        out_shape=jax.ShapeDtypeStruct((N, D), jnp.float32),
    )(partials)
    return out
